# Initial kernel scaffold; baseline (speedup 1.0000x reference)
import jax
import jax.numpy as jnp
from jax.experimental import pallas as pl


def kernel(inputs, W_proj, expert_embeddings, temperature):
    raise NotImplementedError("write your pallas kernel here")


# speedup vs baseline: 1.3461x; 1.3461x over previous
import functools

import jax
import jax.numpy as jnp
from jax.experimental import pallas as pl

NUM_EXPERTS = 64
TOP_K = 2
BN = 1024


def _gating_kernel(x_ref, w_ref, e_ref, t_ref,
                   ew_ref, ei_ref, gl_ref, cs_ref, rp_ref):
    x = x_ref[...]
    w = w_ref[...]
    p = jnp.dot(x, w, preferred_element_type=jnp.float32)
    p_sq = jnp.sum(p * p, axis=-1, keepdims=True)
    p_n = p / jnp.sqrt(jnp.maximum(p_sq, 1e-12))
    e = e_ref[...]
    e_sq = jnp.sum(e * e, axis=0, keepdims=True)
    e_n = e / jnp.sqrt(jnp.maximum(e_sq, 1e-12))
    cos = jnp.dot(p_n, e_n, preferred_element_type=jnp.float32)
    t = t_ref[0, 0]
    gl = cos * t

    iota = jax.lax.broadcasted_iota(jnp.int32, gl.shape, 1)
    m1 = jnp.max(gl, axis=-1, keepdims=True)
    i1 = jnp.min(jnp.where(gl == m1, iota, NUM_EXPERTS), axis=-1, keepdims=True)
    gl_wo1 = jnp.where(iota == i1, -jnp.inf, gl)
    m2 = jnp.max(gl_wo1, axis=-1, keepdims=True)
    i2 = jnp.min(jnp.where(gl_wo1 == m2, iota, NUM_EXPERTS), axis=-1,
                 keepdims=True)
    sel = (iota == i1) | (iota == i2)

    emx = jnp.exp(gl - m1)
    rp = emx / jnp.sum(emx, axis=-1, keepdims=True)
    ew_num = jnp.where(sel, emx, 0.0)
    ew = ew_num / jnp.sum(ew_num, axis=-1, keepdims=True)

    ew_ref[...] = ew
    ei_ref[...] = jnp.concatenate([i1, i2], axis=1)
    gl_ref[...] = gl
    cs_ref[...] = cos
    rp_ref[...] = rp


@jax.jit
def kernel(inputs, W_proj, expert_embeddings, temperature):
    n, d = inputs.shape
    e = NUM_EXPERTS
    grid = (n // BN,)
    t2 = jnp.reshape(temperature.astype(jnp.float32), (1, 1))
    out_shapes = (
        jax.ShapeDtypeStruct((n, e), jnp.float32),
        jax.ShapeDtypeStruct((n, TOP_K), jnp.int32),
        jax.ShapeDtypeStruct((n, e), jnp.float32),
        jax.ShapeDtypeStruct((n, e), jnp.float32),
        jax.ShapeDtypeStruct((n, e), jnp.float32),
    )
    row_spec = pl.BlockSpec((BN, e), lambda i: (i, 0))
    out = pl.pallas_call(
        _gating_kernel,
        grid=grid,
        in_specs=[
            pl.BlockSpec((BN, d), lambda i: (i, 0)),
            pl.BlockSpec((d, e), lambda i: (0, 0)),
            pl.BlockSpec((e, e), lambda i: (0, 0)),
            pl.BlockSpec((1, 1), lambda i: (0, 0)),
        ],
        out_specs=(
            row_spec,
            pl.BlockSpec((BN, TOP_K), lambda i: (i, 0)),
            row_spec,
            row_spec,
            row_spec,
        ),
        out_shape=out_shapes,
    )(inputs, W_proj, expert_embeddings, t2)
    return out


# speedup vs baseline: 1.5257x; 1.1334x over previous
import functools

import jax
import jax.numpy as jnp
from jax.experimental import pallas as pl

NUM_EXPERTS = 64
TOP_K = 2
BN = 4096


def _gating_kernel(x_ref, w_ref, e_ref, t_ref,
                   ew_ref, ei_ref, gl_ref, cs_ref, rp_ref):
    x = x_ref[...]
    w = w_ref[...]
    p = jnp.dot(x, w, preferred_element_type=jnp.float32)
    p_sq = jnp.sum(p * p, axis=-1, keepdims=True)
    p_n = p / jnp.sqrt(jnp.maximum(p_sq, 1e-12))
    e = e_ref[...]
    e_sq = jnp.sum(e * e, axis=0, keepdims=True)
    e_n = e / jnp.sqrt(jnp.maximum(e_sq, 1e-12))
    cos = jnp.dot(p_n, e_n, preferred_element_type=jnp.float32)
    t = t_ref[0, 0]
    gl = cos * t

    iota = jax.lax.broadcasted_iota(jnp.int32, gl.shape, 1)
    m1 = jnp.max(gl, axis=-1, keepdims=True)
    i1 = jnp.min(jnp.where(gl == m1, iota, NUM_EXPERTS), axis=-1, keepdims=True)
    gl_wo1 = jnp.where(iota == i1, -jnp.inf, gl)
    m2 = jnp.max(gl_wo1, axis=-1, keepdims=True)
    i2 = jnp.min(jnp.where(gl_wo1 == m2, iota, NUM_EXPERTS), axis=-1,
                 keepdims=True)
    sel = (iota == i1) | (iota == i2)

    emx = jnp.exp(gl - m1)
    rp = emx / jnp.sum(emx, axis=-1, keepdims=True)
    ew_num = jnp.where(sel, emx, 0.0)
    ew = ew_num / jnp.sum(ew_num, axis=-1, keepdims=True)

    ew_ref[...] = ew
    ei_ref[...] = jnp.concatenate([i1, i2], axis=1)
    gl_ref[...] = gl
    cs_ref[...] = cos
    rp_ref[...] = rp


@jax.jit
def kernel(inputs, W_proj, expert_embeddings, temperature):
    n, d = inputs.shape
    e = NUM_EXPERTS
    grid = (n // BN,)
    t2 = jnp.reshape(temperature.astype(jnp.float32), (1, 1))
    out_shapes = (
        jax.ShapeDtypeStruct((n, e), jnp.float32),
        jax.ShapeDtypeStruct((n, TOP_K), jnp.int32),
        jax.ShapeDtypeStruct((n, e), jnp.float32),
        jax.ShapeDtypeStruct((n, e), jnp.float32),
        jax.ShapeDtypeStruct((n, e), jnp.float32),
    )
    row_spec = pl.BlockSpec((BN, e), lambda i: (i, 0))
    out = pl.pallas_call(
        _gating_kernel,
        grid=grid,
        in_specs=[
            pl.BlockSpec((BN, d), lambda i: (i, 0)),
            pl.BlockSpec((d, e), lambda i: (0, 0)),
            pl.BlockSpec((e, e), lambda i: (0, 0)),
            pl.BlockSpec((1, 1), lambda i: (0, 0)),
        ],
        out_specs=(
            row_spec,
            pl.BlockSpec((BN, TOP_K), lambda i: (i, 0)),
            row_spec,
            row_spec,
            row_spec,
        ),
        out_shape=out_shapes,
    )(inputs, W_proj, expert_embeddings, t2)
    return out


# speedup vs baseline: 1.5707x; 1.0295x over previous
import functools

import jax
import jax.numpy as jnp
from jax.experimental import pallas as pl

NUM_EXPERTS = 64
TOP_K = 2
BN = 4096


def _gating_kernel(x_ref, w_ref, e_ref, t_ref,
                   ew_ref, ei_ref, gl_ref, cs_ref, rp_ref):
    x = x_ref[...]
    w = w_ref[...]
    p = jnp.dot(x, w, preferred_element_type=jnp.float32)
    p_sq = jnp.sum(p * p, axis=-1, keepdims=True)
    p_n = p * (1.0 / jnp.sqrt(jnp.maximum(p_sq, 1e-12)))
    e = e_ref[...]
    e_sq = jnp.sum(e * e, axis=0, keepdims=True)
    e_n = e * (1.0 / jnp.sqrt(jnp.maximum(e_sq, 1e-12)))
    cos = jnp.dot(p_n, e_n, preferred_element_type=jnp.float32)
    t = t_ref[0, 0]
    gl = cos * t

    iota_f = jax.lax.broadcasted_iota(jnp.int32, gl.shape, 1).astype(jnp.float32)
    m1 = jnp.max(gl, axis=-1, keepdims=True)
    i1f = jnp.min(jnp.where(gl == m1, iota_f, jnp.inf), axis=-1, keepdims=True)
    gl_wo1 = jnp.where(iota_f == i1f, -jnp.inf, gl)
    m2 = jnp.max(gl_wo1, axis=-1, keepdims=True)
    i2f = jnp.min(jnp.where(gl_wo1 == m2, iota_f, jnp.inf), axis=-1,
                  keepdims=True)
    sel = (iota_f == i1f) | (iota_f == i2f)

    emx = jnp.exp(gl - m1)
    rp = emx * (1.0 / jnp.sum(emx, axis=-1, keepdims=True))
    ew_num = jnp.where(sel, emx, 0.0)
    ew = ew_num * (1.0 / jnp.sum(ew_num, axis=-1, keepdims=True))

    ew_ref[...] = ew
    ei_ref[...] = jnp.concatenate([i1f.astype(jnp.int32),
                                   i2f.astype(jnp.int32)], axis=1)
    gl_ref[...] = gl
    cs_ref[...] = cos
    rp_ref[...] = rp


@jax.jit
def kernel(inputs, W_proj, expert_embeddings, temperature):
    n, d = inputs.shape
    e = NUM_EXPERTS
    grid = (n // BN,)
    t2 = jnp.reshape(temperature.astype(jnp.float32), (1, 1))
    out_shapes = (
        jax.ShapeDtypeStruct((n, e), jnp.float32),
        jax.ShapeDtypeStruct((n, TOP_K), jnp.int32),
        jax.ShapeDtypeStruct((n, e), jnp.float32),
        jax.ShapeDtypeStruct((n, e), jnp.float32),
        jax.ShapeDtypeStruct((n, e), jnp.float32),
    )
    row_spec = pl.BlockSpec((BN, e), lambda i: (i, 0))
    out = pl.pallas_call(
        _gating_kernel,
        grid=grid,
        in_specs=[
            pl.BlockSpec((BN, d), lambda i: (i, 0)),
            pl.BlockSpec((d, e), lambda i: (0, 0)),
            pl.BlockSpec((e, e), lambda i: (0, 0)),
            pl.BlockSpec((1, 1), lambda i: (0, 0)),
        ],
        out_specs=(
            row_spec,
            pl.BlockSpec((BN, TOP_K), lambda i: (i, 0)),
            row_spec,
            row_spec,
            row_spec,
        ),
        out_shape=out_shapes,
    )(inputs, W_proj, expert_embeddings, t2)
    return out
